# baseline (speedup 1.0000x reference)
import functools

import jax
import jax.numpy as jnp
from jax import lax
from jax.experimental import pallas as pl
from jax.experimental.pallas import tpu as pltpu
from jax.experimental.pallas import tpu_sc as plsc

_N = 10000
_E = 320000
_G = 16
_F0 = 128
_F1 = 128
_F2 = 64

_NC = 2
_NS = 16
_EPT = _E // (_NC * _NS)
_CHUNK40 = 40
_CHUNK80 = 80
_NCHUNK80 = _EPT // _CHUNK80
_NB = 5
_LA = 3
_ZROWS = 640
_NPAD = _NS * _ZROWS

_MESH = plsc.VectorSubcoreMesh(core_axis_name="c", subcore_axis_name="s")


def _sc_degree(dst2):
    zeros = jnp.zeros((_ZROWS,), jnp.float32)
    ones = jnp.ones((_CHUNK80,), jnp.float32)

    @functools.partial(
        pl.kernel,
        out_type=jax.ShapeDtypeStruct((_NC, _NPAD), jnp.float32),
        mesh=_MESH,
        scratch_types=[
            pltpu.VMEM_SHARED((_NPAD,), jnp.float32),
            pltpu.VMEM((_NCHUNK80, _CHUNK80), jnp.int32),
            pltpu.VMEM((_CHUNK80,), jnp.float32),
            pltpu.SemaphoreType.DMA,
        ],
    )
    def deg_kernel(dst_hbm, zeros_hbm, ones_hbm, out_hbm, acc, idx, ones_v,
                   sem):
        c = lax.axis_index("c")
        s = lax.axis_index("s")
        w = c * _NS + s

        pltpu.sync_copy(dst_hbm.at[w], idx)
        pltpu.sync_copy(ones_hbm, ones_v)
        pltpu.sync_copy(zeros_hbm, acc.at[pl.ds(s * _ZROWS, _ZROWS)])
        plsc.subcore_barrier()

        def fire(k, carry):
            pltpu.async_copy(ones_v, acc.at[idx.at[k]], sem, add=True)
            return carry

        lax.fori_loop(0, _NCHUNK80, fire, 0)

        def drain(k, carry):
            pltpu.make_async_copy(ones_v, acc.at[idx.at[k]], sem).wait()
            return carry

        lax.fori_loop(0, _NCHUNK80, drain, 0)
        plsc.subcore_barrier()

        pltpu.sync_copy(acc.at[pl.ds(s * _ZROWS, _ZROWS)],
                        out_hbm.at[c, pl.ds(s * _ZROWS, _ZROWS)])

    return deg_kernel(dst2, zeros, ones)


def _sc_aggregate(hs, src2, dst2, F, chunk):
    nchunk = _EPT // chunk
    zeros = jnp.zeros((_ZROWS, F), jnp.float32)

    @functools.partial(
        pl.kernel,
        out_type=jax.ShapeDtypeStruct((_NC, _NPAD, F), jnp.float32),
        mesh=_MESH,
        compiler_params=pltpu.CompilerParams(use_tc_tiling_on_sc=False),
        scratch_types=[
            pltpu.VMEM_SHARED((_NPAD, F), jnp.float32),
            pltpu.VMEM((nchunk, chunk), jnp.int32),
            pltpu.VMEM((nchunk, chunk), jnp.int32),
            pltpu.VMEM((_NB, chunk, F), jnp.float32),
            pltpu.SemaphoreType.DMA,
            pltpu.SemaphoreType.DMA,
            pltpu.SemaphoreType.DMA,
            pltpu.SemaphoreType.DMA,
            pltpu.SemaphoreType.DMA,
            pltpu.SemaphoreType.DMA,
            pltpu.SemaphoreType.DMA,
            pltpu.SemaphoreType.DMA,
            pltpu.SemaphoreType.DMA,
            pltpu.SemaphoreType.DMA,
        ],
    )
    def agg_kernel(hs_hbm, src_hbm, dst_hbm, zeros_hbm, out_hbm,
                   acc, isrc, idst, rows,
                   gs0, gs1, gs2, gs3, gs4, ss0, ss1, ss2, ss3, ss4):
        c = lax.axis_index("c")
        s = lax.axis_index("s")
        w = c * _NS + s
        gsems = (gs0, gs1, gs2, gs3, gs4)
        ssems = (ss0, ss1, ss2, ss3, ss4)

        pltpu.sync_copy(src_hbm.at[w], isrc)
        pltpu.sync_copy(dst_hbm.at[w], idst)
        pltpu.sync_copy(zeros_hbm, acc.at[pl.ds(s * _ZROWS, _ZROWS)])
        plsc.subcore_barrier()

        def start_gather(k, b):
            pltpu.async_copy(hs_hbm.at[isrc.at[k]], rows.at[b], gsems[b])

        def wait_gather(k, b):
            pltpu.make_async_copy(hs_hbm.at[isrc.at[k]], rows.at[b],
                                  gsems[b]).wait()

        def start_scatter(k, b):
            pltpu.async_copy(rows.at[b], acc.at[idst.at[k]], ssems[b],
                             add=True)

        def wait_scatter(k, b):
            pltpu.make_async_copy(rows.at[b], acc.at[idst.at[k]],
                                  ssems[b]).wait()

        for j in range(_LA):
            start_gather(j, j)

        def slot(k, b):
            @pl.when(k < nchunk)
            def _():
                wait_gather(k, b)
                start_scatter(k, b)
                kn = k + _LA
                @pl.when(kn < nchunk)
                def _():
                    bn = (b + _LA) % _NB
                    @pl.when(kn >= _NB)
                    def _():
                        wait_scatter(kn - _NB, bn)
                    start_gather(kn, bn)

        def body(i, carry):
            for b in range(_NB):
                slot(_NB * i + b, b)
            return carry

        lax.fori_loop(0, (nchunk + _NB - 1) // _NB, body, 0)
        for j in range(_NB):
            k = nchunk - _NB + j
            wait_scatter(k, k % _NB)

        plsc.subcore_barrier()

        pltpu.sync_copy(acc.at[pl.ds(s * _ZROWS, _ZROWS)],
                        out_hbm.at[c, pl.ds(s * _ZROWS, _ZROWS)])

    return agg_kernel(hs, src2, dst2, zeros)


_NRB = 4
_NIB = 6

def _sc_aggregate_ring(hs, ei2):
    F = _F1
    zeros = jnp.zeros((_ZROWS, F), jnp.float32)

    @functools.partial(
        pl.kernel,
        out_type=jax.ShapeDtypeStruct((_NC, _NPAD, F), jnp.float32),
        mesh=_MESH,
        compiler_params=pltpu.CompilerParams(use_tc_tiling_on_sc=False),
        scratch_types=[
            pltpu.VMEM_SHARED((_NPAD, F), jnp.float32),
            pltpu.VMEM((_NIB, 2, _CHUNK80), jnp.int32),
            pltpu.VMEM((_NRB, _CHUNK80, F), jnp.float32),
            pltpu.SemaphoreType.DMA,
            pltpu.SemaphoreType.DMA,
            pltpu.SemaphoreType.DMA,
            pltpu.SemaphoreType.DMA,
            pltpu.SemaphoreType.DMA,
            pltpu.SemaphoreType.DMA,
            pltpu.SemaphoreType.DMA,
            pltpu.SemaphoreType.DMA,
            pltpu.SemaphoreType.DMA,
            pltpu.SemaphoreType.DMA,
            pltpu.SemaphoreType.DMA,
            pltpu.SemaphoreType.DMA,
            pltpu.SemaphoreType.DMA,
            pltpu.SemaphoreType.DMA,
        ],
    )
    def agg_kernel(hs_hbm, ei_hbm, zeros_hbm, out_hbm,
                   acc, ibuf, rows,
                   is0, is1, is2, is3, is4, is5,
                   gs0, gs1, gs2, gs3, ss0, ss1, ss2, ss3):
        c = lax.axis_index("c")
        s = lax.axis_index("s")
        w = c * _NS + s
        isems = (is0, is1, is2, is3, is4, is5)
        gsems = (gs0, gs1, gs2, gs3)
        ssems = (ss0, ss1, ss2, ss3)
        NCH = _NCHUNK80

        pltpu.sync_copy(zeros_hbm, acc.at[pl.ds(s * _ZROWS, _ZROWS)])

        def fire_idx(k, bi):
            pltpu.async_copy(ei_hbm.at[w, k], ibuf.at[bi], isems[bi])

        def wait_idx(k, bi):
            pltpu.make_async_copy(ei_hbm.at[w, k], ibuf.at[bi],
                                  isems[bi]).wait()

        def start_gather(k, b, bi):
            pltpu.async_copy(hs_hbm.at[ibuf.at[bi, 0]], rows.at[b], gsems[b])

        def wait_gather(k, b, bi):
            pltpu.make_async_copy(hs_hbm.at[ibuf.at[bi, 0]], rows.at[b],
                                  gsems[b]).wait()

        def start_scatter(k, b, bi):
            pltpu.async_copy(rows.at[b], acc.at[ibuf.at[bi, 1]], ssems[b],
                             add=True)

        def wait_scatter(k, b, bi):
            pltpu.make_async_copy(rows.at[b], acc.at[ibuf.at[bi, 1]],
                                  ssems[b]).wait()

        for t in range(_NIB):
            fire_idx(t, t)
        plsc.subcore_barrier()
        for t in range(2):
            wait_idx(t, t)
            start_gather(t, t, t)

        def slot(k, t):
            b = t % _NRB
            bi = t % _NIB

            @pl.when(k < NCH)
            def _():
                wait_gather(k, b, bi)
                start_scatter(k, b, bi)
                kn = k + 2
                @pl.when(kn < NCH)
                def _():
                    bn = (t + 2) % _NRB
                    bin_ = (t + 2) % _NIB
                    wait_idx(kn, bin_)
                    @pl.when(kn >= 4)
                    def _():
                        wait_scatter(k - 2, bn, (t + 4) % _NIB)
                        @pl.when(k + 4 < NCH)
                        def _():
                            fire_idx(k + 4, (t + 4) % _NIB)
                    start_gather(kn, bn, bin_)

        nslots = 12
        def body(i, carry):
            for t in range(nslots):
                slot(nslots * i + t, t)
            return carry

        lax.fori_loop(0, (NCH + nslots - 1) // nslots, body, 0)
        for j in range(_NRB):
            k = NCH - _NRB + j
            wait_scatter(k, k % _NRB, k % _NIB)

        plsc.subcore_barrier()

        pltpu.sync_copy(acc.at[pl.ds(s * _ZROWS, _ZROWS)],
                        out_hbm.at[c, pl.ds(s * _ZROWS, _ZROWS)])

    return agg_kernel(hs, ei2, zeros)


_BN = 2000


def _tc_prescale(x, W1, cnt0, cnt1):
    def body(c0_ref, c1_ref, x_ref, w_ref, hs_ref, dv_ref):
        dv = lax.rsqrt(c0_ref[...] + c1_ref[...] + 1.0)
        dv_ref[...] = dv
        h = jnp.dot(x_ref[...], w_ref[...], preferred_element_type=jnp.float32)
        hs_ref[...] = h * dv

    return pl.pallas_call(
        body,
        grid=(_N // _BN,),
        in_specs=[
            pl.BlockSpec((_BN, 1), lambda i: (i, 0)),
            pl.BlockSpec((_BN, 1), lambda i: (i, 0)),
            pl.BlockSpec((_BN, _F0), lambda i: (i, 0)),
            pl.BlockSpec((_F0, _F1), lambda i: (0, 0)),
        ],
        out_specs=[
            pl.BlockSpec((_BN, _F1), lambda i: (i, 0)),
            pl.BlockSpec((_BN, 1), lambda i: (i, 0)),
        ],
        out_shape=[
            jax.ShapeDtypeStruct((_N, _F1), jnp.float32),
            jax.ShapeDtypeStruct((_N, 1), jnp.float32),
        ],
    )(cnt0, cnt1, x, W1)


def _tc_mid(ys1, hs1, dinv, b1, W2):
    def body(ys_ref, hs_ref, dv_ref, b1_ref, w2_ref, out_ref):
        dv = dv_ref[...]
        t = (ys_ref[0] + ys_ref[1] + hs_ref[...]) * dv + b1_ref[...]
        h1 = jnp.maximum(t, 0.0)
        out_ref[...] = jnp.dot(h1, w2_ref[...],
                               preferred_element_type=jnp.float32) * dv

    return pl.pallas_call(
        body,
        grid=(_N // _BN,),
        in_specs=[
            pl.BlockSpec((_NC, _BN, _F1), lambda i: (0, i, 0)),
            pl.BlockSpec((_BN, _F1), lambda i: (i, 0)),
            pl.BlockSpec((_BN, 1), lambda i: (i, 0)),
            pl.BlockSpec((1, _F1), lambda i: (0, 0)),
            pl.BlockSpec((_F1, _F2), lambda i: (0, 0)),
        ],
        out_specs=pl.BlockSpec((_BN, _F2), lambda i: (i, 0)),
        out_shape=jax.ShapeDtypeStruct((_N, _F2), jnp.float32),
    )(ys1, hs1, dinv, b1.reshape(1, _F1), W2)


def _tc_readout(ys2, hs2, dinv, b2, batch, Wm, bm):
    ngrid = _N // _BN

    def body(ys_ref, hs_ref, dv_ref, b2_ref, bat_ref, wmean_ref, wmax_ref,
             bm_ref, out_ref, sum_acc, max_acc, cnt_acc):
        i = pl.program_id(0)

        @pl.when(i == 0)
        def _():
            sum_acc[...] = jnp.zeros_like(sum_acc)
            max_acc[...] = jnp.full_like(max_acc, -jnp.inf)
            cnt_acc[...] = jnp.zeros_like(cnt_acc)

        h2 = (ys_ref[0] + ys_ref[1] + hs_ref[...]) * dv_ref[...] + b2_ref[...]
        bat = bat_ref[...]
        bmin = bat_ref[0, 0]
        bmax = bat_ref[_BN - 1, 0]

        for g in range(_G):
            @pl.when((bmin <= g) & (g <= bmax))
            def _():
                m = bat == g
                sum_acc[g:g + 1, :] += jnp.sum(
                    jnp.where(m, h2, 0.0), axis=0, keepdims=True)
                max_acc[g:g + 1, :] = jnp.maximum(
                    max_acc[g:g + 1, :],
                    jnp.max(jnp.where(m, h2, -jnp.inf), axis=0, keepdims=True))
                cnt_acc[g:g + 1, :] += jnp.sum(
                    m.astype(jnp.float32), axis=0, keepdims=True)

        @pl.when(i == ngrid - 1)
        def _():
            mean = sum_acc[...] / jnp.maximum(cnt_acc[...], 1.0)
            out_ref[...] = (
                jnp.dot(mean, wmean_ref[...], preferred_element_type=jnp.float32)
                + jnp.dot(max_acc[...], wmax_ref[...],
                          preferred_element_type=jnp.float32)
                + bm_ref[...])

    return pl.pallas_call(
        body,
        grid=(ngrid,),
        in_specs=[
            pl.BlockSpec((_NC, _BN, _F2), lambda i: (0, i, 0)),
            pl.BlockSpec((_BN, _F2), lambda i: (i, 0)),
            pl.BlockSpec((_BN, 1), lambda i: (i, 0)),
            pl.BlockSpec((1, _F2), lambda i: (0, 0)),
            pl.BlockSpec((_BN, 1), lambda i: (i, 0)),
            pl.BlockSpec((_F2, 2), lambda i: (0, 0)),
            pl.BlockSpec((_F2, 2), lambda i: (0, 0)),
            pl.BlockSpec((1, 2), lambda i: (0, 0)),
        ],
        out_specs=pl.BlockSpec((_G, 2), lambda i: (0, 0)),
        out_shape=jax.ShapeDtypeStruct((_G, 2), jnp.float32),
        scratch_shapes=[
            pltpu.VMEM((_G, _F2), jnp.float32),
            pltpu.VMEM((_G, _F2), jnp.float32),
            pltpu.VMEM((_G, 1), jnp.float32),
        ],
    )(ys2, hs2, dinv, b2.reshape(1, _F2), batch.reshape(_N, 1),
      Wm[:_F2], Wm[_F2:], bm.reshape(1, 2))


def kernel(x, edge_index, batch, W1, b1, W2, b2, Wm, bm):
    ei2 = edge_index.reshape(2, _NC * _NS, _NCHUNK80, _CHUNK80).transpose(
        1, 2, 0, 3)
    src80 = edge_index[0].reshape(_NC * _NS, _NCHUNK80, _CHUNK80)
    dst80 = edge_index[1].reshape(_NC * _NS, _NCHUNK80, _CHUNK80)
    cnt = _sc_degree(dst80)
    cnt0 = cnt[0, :_N].reshape(_N, 1)
    cnt1 = cnt[1, :_N].reshape(_N, 1)
    hs1, dinv = _tc_prescale(x, W1, cnt0, cnt1)
    ys1 = _sc_aggregate_ring(hs1, ei2)
    hs2 = _tc_mid(ys1, hs1, dinv, b1, W2)
    ys2 = _sc_aggregate(hs2, src80, dst80, _F2, _CHUNK80)
    return _tc_readout(ys2, hs2, dinv, b2, batch, Wm, bm)


# speedup vs baseline: 1.0917x; 1.0917x over previous
import functools

import jax
import jax.numpy as jnp
from jax import lax
from jax.experimental import pallas as pl
from jax.experimental.pallas import tpu as pltpu
from jax.experimental.pallas import tpu_sc as plsc

_N = 10000
_E = 320000
_G = 16
_F0 = 128
_F1 = 128
_F2 = 64

_NC = 2
_NS = 16
_EPT = _E // (_NC * _NS)
_CHUNK40 = 40
_CHUNK80 = 80
_NCHUNK80 = _EPT // _CHUNK80
_NB = 5
_LA = 3
_ZROWS = 640
_NPAD = _NS * _ZROWS

_MESH = plsc.VectorSubcoreMesh(core_axis_name="c", subcore_axis_name="s")


def _sc_degree(dst2):
    zeros = jnp.zeros((_ZROWS,), jnp.float32)
    ones = jnp.ones((_CHUNK80,), jnp.float32)

    @functools.partial(
        pl.kernel,
        out_type=jax.ShapeDtypeStruct((_NC, _NPAD), jnp.float32),
        mesh=_MESH,
        scratch_types=[
            pltpu.VMEM_SHARED((_NPAD,), jnp.float32),
            pltpu.VMEM((_NCHUNK80, _CHUNK80), jnp.int32),
            pltpu.VMEM((_CHUNK80,), jnp.float32),
            pltpu.SemaphoreType.DMA,
        ],
    )
    def deg_kernel(dst_hbm, zeros_hbm, ones_hbm, out_hbm, acc, idx, ones_v,
                   sem):
        c = lax.axis_index("c")
        s = lax.axis_index("s")
        w = c * _NS + s

        pltpu.sync_copy(dst_hbm.at[w], idx)
        pltpu.sync_copy(ones_hbm, ones_v)
        pltpu.sync_copy(zeros_hbm, acc.at[pl.ds(s * _ZROWS, _ZROWS)])
        plsc.subcore_barrier()

        def fire(k, carry):
            pltpu.async_copy(ones_v, acc.at[idx.at[k]], sem, add=True)
            return carry

        lax.fori_loop(0, _NCHUNK80, fire, 0)

        def drain(k, carry):
            pltpu.make_async_copy(ones_v, acc.at[idx.at[k]], sem).wait()
            return carry

        lax.fori_loop(0, _NCHUNK80, drain, 0)
        plsc.subcore_barrier()

        pltpu.sync_copy(acc.at[pl.ds(s * _ZROWS, _ZROWS)],
                        out_hbm.at[c, pl.ds(s * _ZROWS, _ZROWS)])

    return deg_kernel(dst2, zeros, ones)


def _sc_aggregate(hs, src2, dst2, F, chunk):
    nchunk = _EPT // chunk
    zeros = jnp.zeros((_ZROWS, F), jnp.float32)

    @functools.partial(
        pl.kernel,
        out_type=jax.ShapeDtypeStruct((_NC, _NPAD, F), jnp.float32),
        mesh=_MESH,
        compiler_params=pltpu.CompilerParams(use_tc_tiling_on_sc=False),
        scratch_types=[
            pltpu.VMEM_SHARED((_NPAD, F), jnp.float32),
            pltpu.VMEM((nchunk, chunk), jnp.int32),
            pltpu.VMEM((nchunk, chunk), jnp.int32),
            pltpu.VMEM((_NB, chunk, F), jnp.float32),
            pltpu.SemaphoreType.DMA,
            pltpu.SemaphoreType.DMA,
            pltpu.SemaphoreType.DMA,
            pltpu.SemaphoreType.DMA,
            pltpu.SemaphoreType.DMA,
            pltpu.SemaphoreType.DMA,
            pltpu.SemaphoreType.DMA,
            pltpu.SemaphoreType.DMA,
            pltpu.SemaphoreType.DMA,
            pltpu.SemaphoreType.DMA,
        ],
    )
    def agg_kernel(hs_hbm, src_hbm, dst_hbm, zeros_hbm, out_hbm,
                   acc, isrc, idst, rows,
                   gs0, gs1, gs2, gs3, gs4, ss0, ss1, ss2, ss3, ss4):
        c = lax.axis_index("c")
        s = lax.axis_index("s")
        w = c * _NS + s
        gsems = (gs0, gs1, gs2, gs3, gs4)
        ssems = (ss0, ss1, ss2, ss3, ss4)

        pltpu.sync_copy(src_hbm.at[w], isrc)
        pltpu.sync_copy(dst_hbm.at[w], idst)

        def start_gather(k, b):
            pltpu.async_copy(hs_hbm.at[isrc.at[k]], rows.at[b], gsems[b])

        def wait_gather(k, b):
            pltpu.make_async_copy(hs_hbm.at[isrc.at[k]], rows.at[b],
                                  gsems[b]).wait()

        def start_scatter(k, b):
            pltpu.async_copy(rows.at[b], acc.at[idst.at[k]], ssems[b],
                             add=True)

        def wait_scatter(k, b):
            pltpu.make_async_copy(rows.at[b], acc.at[idst.at[k]],
                                  ssems[b]).wait()

        for j in range(_LA):
            start_gather(j, j)
        pltpu.sync_copy(zeros_hbm, acc.at[pl.ds(s * _ZROWS, _ZROWS)])
        plsc.subcore_barrier()

        def slot(k, b):
            @pl.when(k < nchunk)
            def _():
                wait_gather(k, b)
                start_scatter(k, b)
                kn = k + _LA
                @pl.when(kn < nchunk)
                def _():
                    bn = (b + _LA) % _NB
                    @pl.when(kn >= _NB)
                    def _():
                        wait_scatter(kn - _NB, bn)
                    start_gather(kn, bn)

        def body(i, carry):
            for b in range(_NB):
                slot(_NB * i + b, b)
            return carry

        lax.fori_loop(0, (nchunk + _NB - 1) // _NB, body, 0)
        for j in range(_NB):
            k = nchunk - _NB + j
            wait_scatter(k, k % _NB)

        plsc.subcore_barrier()

        pltpu.sync_copy(acc.at[pl.ds(s * _ZROWS, _ZROWS)],
                        out_hbm.at[c, pl.ds(s * _ZROWS, _ZROWS)])

    return agg_kernel(hs, src2, dst2, zeros)


_BN = 2000


def _tc_prescale(x, W1, cnt0, cnt1):
    def body(c0_ref, c1_ref, x_ref, w_ref, hs_ref, dv_ref):
        dv = lax.rsqrt(c0_ref[...] + c1_ref[...] + 1.0)
        dv_ref[...] = dv
        h = jnp.dot(x_ref[...], w_ref[...], preferred_element_type=jnp.float32)
        hs_ref[...] = h * dv

    return pl.pallas_call(
        body,
        grid=(_N // _BN,),
        in_specs=[
            pl.BlockSpec((_BN, 1), lambda i: (i, 0)),
            pl.BlockSpec((_BN, 1), lambda i: (i, 0)),
            pl.BlockSpec((_BN, _F0), lambda i: (i, 0)),
            pl.BlockSpec((_F0, _F1), lambda i: (0, 0)),
        ],
        out_specs=[
            pl.BlockSpec((_BN, _F1), lambda i: (i, 0)),
            pl.BlockSpec((_BN, 1), lambda i: (i, 0)),
        ],
        out_shape=[
            jax.ShapeDtypeStruct((_N, _F1), jnp.float32),
            jax.ShapeDtypeStruct((_N, 1), jnp.float32),
        ],
    )(cnt0, cnt1, x, W1)


def _tc_mid(ys1, hs1, dinv, b1, W2):
    def body(ys_ref, hs_ref, dv_ref, b1_ref, w2_ref, out_ref):
        dv = dv_ref[...]
        t = (ys_ref[0] + ys_ref[1] + hs_ref[...]) * dv + b1_ref[...]
        h1 = jnp.maximum(t, 0.0)
        out_ref[...] = jnp.dot(h1, w2_ref[...],
                               preferred_element_type=jnp.float32) * dv

    return pl.pallas_call(
        body,
        grid=(_N // _BN,),
        in_specs=[
            pl.BlockSpec((_NC, _BN, _F1), lambda i: (0, i, 0)),
            pl.BlockSpec((_BN, _F1), lambda i: (i, 0)),
            pl.BlockSpec((_BN, 1), lambda i: (i, 0)),
            pl.BlockSpec((1, _F1), lambda i: (0, 0)),
            pl.BlockSpec((_F1, _F2), lambda i: (0, 0)),
        ],
        out_specs=pl.BlockSpec((_BN, _F2), lambda i: (i, 0)),
        out_shape=jax.ShapeDtypeStruct((_N, _F2), jnp.float32),
    )(ys1, hs1, dinv, b1.reshape(1, _F1), W2)


def _tc_readout(ys2, hs2, dinv, b2, batch, Wm, bm):
    ngrid = _N // _BN

    def body(ys_ref, hs_ref, dv_ref, b2_ref, bat_ref, wmean_ref, wmax_ref,
             bm_ref, out_ref, sum_acc, max_acc, cnt_acc):
        i = pl.program_id(0)

        @pl.when(i == 0)
        def _():
            sum_acc[...] = jnp.zeros_like(sum_acc)
            max_acc[...] = jnp.full_like(max_acc, -jnp.inf)
            cnt_acc[...] = jnp.zeros_like(cnt_acc)

        h2 = (ys_ref[0] + ys_ref[1] + hs_ref[...]) * dv_ref[...] + b2_ref[...]
        bat = bat_ref[...]
        bmin = bat_ref[0, 0]
        bmax = bat_ref[_BN - 1, 0]

        for g in range(_G):
            @pl.when((bmin <= g) & (g <= bmax))
            def _():
                m = bat == g
                sum_acc[g:g + 1, :] += jnp.sum(
                    jnp.where(m, h2, 0.0), axis=0, keepdims=True)
                max_acc[g:g + 1, :] = jnp.maximum(
                    max_acc[g:g + 1, :],
                    jnp.max(jnp.where(m, h2, -jnp.inf), axis=0, keepdims=True))
                cnt_acc[g:g + 1, :] += jnp.sum(
                    m.astype(jnp.float32), axis=0, keepdims=True)

        @pl.when(i == ngrid - 1)
        def _():
            mean = sum_acc[...] / jnp.maximum(cnt_acc[...], 1.0)
            out_ref[...] = (
                jnp.dot(mean, wmean_ref[...], preferred_element_type=jnp.float32)
                + jnp.dot(max_acc[...], wmax_ref[...],
                          preferred_element_type=jnp.float32)
                + bm_ref[...])

    return pl.pallas_call(
        body,
        grid=(ngrid,),
        in_specs=[
            pl.BlockSpec((_NC, _BN, _F2), lambda i: (0, i, 0)),
            pl.BlockSpec((_BN, _F2), lambda i: (i, 0)),
            pl.BlockSpec((_BN, 1), lambda i: (i, 0)),
            pl.BlockSpec((1, _F2), lambda i: (0, 0)),
            pl.BlockSpec((_BN, 1), lambda i: (i, 0)),
            pl.BlockSpec((_F2, 2), lambda i: (0, 0)),
            pl.BlockSpec((_F2, 2), lambda i: (0, 0)),
            pl.BlockSpec((1, 2), lambda i: (0, 0)),
        ],
        out_specs=pl.BlockSpec((_G, 2), lambda i: (0, 0)),
        out_shape=jax.ShapeDtypeStruct((_G, 2), jnp.float32),
        scratch_shapes=[
            pltpu.VMEM((_G, _F2), jnp.float32),
            pltpu.VMEM((_G, _F2), jnp.float32),
            pltpu.VMEM((_G, 1), jnp.float32),
        ],
    )(ys2, hs2, dinv, b2.reshape(1, _F2), batch.reshape(_N, 1),
      Wm[:_F2], Wm[_F2:], bm.reshape(1, 2))


def kernel(x, edge_index, batch, W1, b1, W2, b2, Wm, bm):
    src40 = edge_index[0].reshape(_NC * _NS, _EPT // _CHUNK40, _CHUNK40)
    dst40 = edge_index[1].reshape(_NC * _NS, _EPT // _CHUNK40, _CHUNK40)
    src80 = edge_index[0].reshape(_NC * _NS, _NCHUNK80, _CHUNK80)
    dst80 = edge_index[1].reshape(_NC * _NS, _NCHUNK80, _CHUNK80)
    cnt = _sc_degree(dst80)
    cnt0 = cnt[0, :_N].reshape(_N, 1)
    cnt1 = cnt[1, :_N].reshape(_N, 1)
    hs1, dinv = _tc_prescale(x, W1, cnt0, cnt1)
    ys1 = _sc_aggregate(hs1, src40, dst40, _F1, _CHUNK40)
    hs2 = _tc_mid(ys1, hs1, dinv, b1, W2)
    ys2 = _sc_aggregate(hs2, src80, dst80, _F2, _CHUNK80)
    return _tc_readout(ys2, hs2, dinv, b2, batch, Wm, bm)
